# Initial kernel scaffold; baseline (speedup 1.0000x reference)
import jax
import jax.numpy as jnp
from jax.experimental import pallas as pl


def kernel(x):
    raise NotImplementedError("write your pallas kernel here")


# speedup vs baseline: 3.0465x; 3.0465x over previous
import jax
import jax.numpy as jnp
from jax import lax
from jax.experimental import pallas as pl
from jax.experimental.pallas import tpu as pltpu

_DIM = 52
_S = _DIM * _DIM
_C = 85
_STRIDE = 416.0 / _DIM
_AW = (373.0 / _STRIDE, 156.0 / _STRIDE, 116.0 / _STRIDE)
_AH = (326.0 / _STRIDE, 198.0 / _STRIDE, 90.0 / _STRIDE)


def _decode_body(x_ref, o_ref):
    a = pl.program_id(1)
    v = x_ref[0, 0]

    row = lax.broadcasted_iota(jnp.int32, (_C, _S), 0)
    col = lax.broadcasted_iota(jnp.int32, (_C, _S), 1)
    gx = (col % _DIM).astype(jnp.float32)
    gy = (col // _DIM).astype(jnp.float32)

    is_exp = (row == 2) | (row == 3)
    e = jnp.exp(jnp.where(is_exp, v, -v))
    sig = 1.0 / (1.0 + e)

    aw = jnp.where(a == 0, _AW[0], jnp.where(a == 1, _AW[1], _AW[2]))
    ah = jnp.where(a == 0, _AH[0], jnp.where(a == 1, _AH[1], _AH[2]))
    scale = jnp.where(row == 2, aw, ah)

    dec = jnp.where(is_exp, e * scale, sig)
    dec = jnp.where(row == 0, dec + gx, dec)
    dec = jnp.where(row == 1, dec + gy, dec)

    o_ref[0, 0] = dec.T


def kernel(x):
    B = x.shape[0]
    x4 = x.reshape(B, 3, _C, _S)
    out = pl.pallas_call(
        _decode_body,
        grid=(B, 3),
        in_specs=[pl.BlockSpec((1, 1, _C, _S), lambda b, a: (b, a, 0, 0))],
        out_specs=pl.BlockSpec((1, 1, _S, _C), lambda b, a: (b, a, 0, 0)),
        out_shape=jax.ShapeDtypeStruct((B, 3, _S, _C), jnp.float32),
    )(x4)
    return out.reshape(B, 3, _DIM, _DIM, _C)


# speedup vs baseline: 3.2592x; 1.0698x over previous
import jax
import jax.numpy as jnp
from jax import lax
from jax.experimental import pallas as pl
from jax.experimental.pallas import tpu as pltpu

_DIM = 52
_S = _DIM * _DIM
_C = 85
_STRIDE = 416.0 / _DIM
_AW = (373.0 / _STRIDE, 156.0 / _STRIDE, 116.0 / _STRIDE)
_AH = (326.0 / _STRIDE, 198.0 / _STRIDE, 90.0 / _STRIDE)


def _decode_body(x_ref, o_ref):
    row = lax.broadcasted_iota(jnp.int32, (_C, _S), 0)
    col = lax.broadcasted_iota(jnp.int32, (_C, _S), 1)
    gx = (col % _DIM).astype(jnp.float32)
    gy = (col // _DIM).astype(jnp.float32)
    is_exp = (row == 2) | (row == 3)

    for a in range(3):
        v = x_ref[0, a]
        e = jnp.exp(jnp.where(is_exp, v, -v))
        sig = 1.0 / (1.0 + e)

        scale = jnp.where(row == 2, _AW[a], _AH[a])
        dec = jnp.where(is_exp, e * scale, sig)
        dec = jnp.where(row == 0, dec + gx, dec)
        dec = jnp.where(row == 1, dec + gy, dec)

        o_ref[0, a] = dec.T


def kernel(x):
    B = x.shape[0]
    x4 = x.reshape(B, 3, _C, _S)
    out = pl.pallas_call(
        _decode_body,
        grid=(B,),
        in_specs=[pl.BlockSpec((1, 3, _C, _S), lambda b: (b, 0, 0, 0))],
        out_specs=pl.BlockSpec((1, 3, _S, _C), lambda b: (b, 0, 0, 0)),
        out_shape=jax.ShapeDtypeStruct((B, 3, _S, _C), jnp.float32),
    )(x4)
    return out.reshape(B, 3, _DIM, _DIM, _C)


# speedup vs baseline: 3.2783x; 1.0059x over previous
import jax
import jax.numpy as jnp
from jax import lax
from jax.experimental import pallas as pl
from jax.experimental.pallas import tpu as pltpu

_DIM = 52
_S = _DIM * _DIM
_C = 85
_STRIDE = 416.0 / _DIM
_AW = (373.0 / _STRIDE, 156.0 / _STRIDE, 116.0 / _STRIDE)
_AH = (326.0 / _STRIDE, 198.0 / _STRIDE, 90.0 / _STRIDE)
_BB = 2


def _decode_body(x_ref, o_ref):
    row = lax.broadcasted_iota(jnp.int32, (_C, _S), 0)
    col = lax.broadcasted_iota(jnp.int32, (_C, _S), 1)
    gx = (col % _DIM).astype(jnp.float32)
    gy = (col // _DIM).astype(jnp.float32)
    is_exp = (row == 2) | (row == 3)

    for b in range(_BB):
        for a in range(3):
            v = x_ref[b, a]
            e = jnp.exp(jnp.where(is_exp, v, -v))
            sig = 1.0 / (1.0 + e)

            scale = jnp.where(row == 2, _AW[a], _AH[a])
            dec = jnp.where(is_exp, e * scale, sig)
            dec = jnp.where(row == 0, dec + gx, dec)
            dec = jnp.where(row == 1, dec + gy, dec)

            o_ref[b, a] = dec.T


def kernel(x):
    B = x.shape[0]
    x4 = x.reshape(B, 3, _C, _S)
    out = pl.pallas_call(
        _decode_body,
        grid=(B // _BB,),
        in_specs=[pl.BlockSpec((_BB, 3, _C, _S), lambda b: (b, 0, 0, 0))],
        out_specs=pl.BlockSpec((_BB, 3, _S, _C), lambda b: (b, 0, 0, 0)),
        out_shape=jax.ShapeDtypeStruct((B, 3, _S, _C), jnp.float32),
    )(x4)
    return out.reshape(B, 3, _DIM, _DIM, _C)
